# Initial kernel scaffold; baseline (speedup 1.0000x reference)
import jax
import jax.numpy as jnp
from jax.experimental import pallas as pl


def kernel(feture, scores, target):
    raise NotImplementedError("write your pallas kernel here")


# speedup vs baseline: 31.0664x; 31.0664x over previous
import functools

import jax
import jax.numpy as jnp
from jax.experimental import pallas as pl

_N = 1024
_C = 128
_NUM_CLASSES = 100
_K = 20
_LAM = 50.0
_BLK = 128
_NBLK = _N // _BLK
_BIG = 3e38


def _dlp_kernel(f_ref, ft_ref, sc_ref, tcol_ref, trow_ref, out_ref):
    f = f_ref[...]
    ft = ft_ref[...]
    tcol = tcol_ref[...]
    trow = trow_ref[...]

    q_row = jnp.sum(ft * ft, axis=0, keepdims=True)

    ciota_col = jax.lax.broadcasted_iota(jnp.int32, (_C, 1), 0)
    ohT = (ciota_col == trow).astype(jnp.float32)
    cs = jax.lax.dot(ohT, f, precision=jax.lax.Precision.HIGHEST)
    cc_col = jnp.sum(ohT, axis=1, keepdims=True)
    cq_col = jnp.sum(ohT * q_row, axis=1, keepdims=True)

    sc = sc_ref[...]
    ciota_row = jax.lax.broadcasted_iota(jnp.int32, (1, _C), 1)
    oh_full = tcol == ciota_row
    smax = jnp.max(sc, axis=1, keepdims=True)
    lse = jnp.log(jnp.sum(jnp.exp(sc - smax), axis=1, keepdims=True)) + smax
    s_t = jnp.sum(jnp.where(oh_full, sc, 0.0), axis=1, keepdims=True)
    ce = jnp.sum(lse - s_t) * (1.0 / _N)

    def block_body(blk, acc):
        r0 = blk * _BLK
        fi = f_ref[pl.ds(r0, _BLK), :]
        ti = tcol_ref[pl.ds(r0, _BLK), :]
        oh_i = (ti == ciota_row).astype(jnp.float32)
        cnt = (
            jax.lax.dot(oh_i, cc_col, precision=jax.lax.Precision.HIGHEST)
            - 1.0
        )
        m = jnp.minimum(cnt, jnp.float32(_K))
        mm = jnp.maximum(m, 1.0)
        qi = jnp.sum(fi * fi, axis=1, keepdims=True)

        has_big = jnp.any(cnt > jnp.float32(_K))

        def closed_path(_):
            s_sel = (
                jax.lax.dot(oh_i, cs, precision=jax.lax.Precision.HIGHEST) - fi
            )
            q_sel = (
                jax.lax.dot(oh_i, cq_col, precision=jax.lax.Precision.HIGHEST)
                - qi
            )
            return s_sel, q_sel

        def general_path(_):
            parts = []
            for jc in range(_NBLK):
                ftc = ft_ref[:, jc * _BLK:(jc + 1) * _BLK]
                diff = jnp.abs(fi[:, :, None] - ftc[None, :, :] + 1e-6)
                parts.append(jnp.sum(diff, axis=1))
            d = jnp.concatenate(parts, axis=1)
            rows = r0 + jax.lax.broadcasted_iota(jnp.int32, (_BLK, 1), 0)
            cols = jax.lax.broadcasted_iota(jnp.int32, (_BLK, _N), 1)
            same = (ti == trow) & (rows != cols)
            dm = jnp.where(same, d, jnp.float32(_BIG))

            def step(k, carry):
                dw, w = carry
                v = jnp.min(dw, axis=1, keepdims=True)
                ismin = dw == v
                jmin = jnp.min(
                    jnp.where(ismin, cols, jnp.int32(2**30)),
                    axis=1, keepdims=True,
                )
                onehot = cols == jmin
                sel = k.astype(jnp.float32) < m
                w = w + jnp.where(onehot & sel, 1.0, 0.0)
                dw = jnp.where(onehot, jnp.float32(_BIG), dw)
                return dw, w

            _, w = jax.lax.fori_loop(
                0, _K, step, (dm, jnp.zeros((_BLK, _N), jnp.float32))
            )
            s_sel = jax.lax.dot(w, f, precision=jax.lax.Precision.HIGHEST)
            q_sel = jnp.sum(w * q_row, axis=1, keepdims=True)
            return s_sel, q_sel

        s_sel, q_sel = jax.lax.cond(has_big, general_path, closed_path, None)
        contrib = (
            m * qi
            - (2.0 / mm) * jnp.sum(fi * s_sel, axis=1, keepdims=True)
            + q_sel / (mm * mm)
        )
        return acc + jnp.sum(contrib)

    knn = jax.lax.fori_loop(0, _NBLK, block_body, jnp.float32(0.0))
    total = ce + (_LAM * 0.5) * knn
    out_ref[...] = jnp.full((1, 1), total, dtype=jnp.float32)


@jax.jit
def kernel(feture, scores, target):
    f = feture.astype(jnp.float32)
    ft = f.T
    t32 = target.astype(jnp.int32)
    tcol = t32.reshape(_N, 1)
    trow = t32.reshape(1, _N)
    sc_pad = jnp.pad(
        scores.astype(jnp.float32),
        ((0, 0), (0, _C - _NUM_CLASSES)),
        constant_values=-1e30,
    )
    out = pl.pallas_call(
        _dlp_kernel,
        out_shape=jax.ShapeDtypeStruct((1, 1), jnp.float32),
    )(f, ft, sc_pad, tcol, trow)
    return out[0, 0]
